# baseline (speedup 1.0000x reference)
import functools

import jax
import jax.numpy as jnp
from jax import lax
from jax.experimental import pallas as pl
from jax.experimental.pallas import tpu as pltpu
from jax.experimental.pallas import tpu_sc as plsc


def _tile_kernel(row_ref, col_ref, out_ref):
    h = row_ref.shape[0]
    w = col_ref.shape[0]
    f_half = row_ref.shape[1]
    out_ref[:, :, 0:f_half] = jnp.broadcast_to(col_ref[...][None, :, :], (h, w, f_half))
    out_ref[:, :, f_half:2 * f_half] = jnp.broadcast_to(row_ref[...][:, None, :], (h, w, f_half))


def kernel(mask, row_embed, col_embed):
    b, h, w = mask.shape
    f_half = row_embed.shape[1]
    f = 2 * f_half

    tile = pl.pallas_call(
        _tile_kernel,
        out_shape=jax.ShapeDtypeStruct((h, w, f), jnp.float32),
    )(row_embed, col_embed)

    info = plsc.get_sparse_core_info()
    nw = info.num_cores * info.num_subcores
    rows_per = h // nw

    @functools.partial(
        pl.kernel,
        out_type=jax.ShapeDtypeStruct((b, h, w, f), jnp.float32),
        mesh=plsc.VectorSubcoreMesh(core_axis_name="c", subcore_axis_name="s"),
        scratch_types=[
            pltpu.VMEM((rows_per, w, f), jnp.float32),
            pltpu.SemaphoreType.DMA,
        ],
    )
    def _replicate(tile_hbm, out_hbm, slice_v, sem):
        wid = lax.axis_index("s") * info.num_cores + lax.axis_index("c")
        base = wid * rows_per
        pltpu.sync_copy(tile_hbm.at[pl.ds(base, rows_per)], slice_v)
        copies = [
            pltpu.make_async_copy(slice_v, out_hbm.at[i, pl.ds(base, rows_per)], sem)
            for i in range(b)
        ]
        for c in copies:
            c.start()
        for c in copies:
            c.wait()

    out = _replicate(tile)
    return jnp.transpose(out, (0, 3, 1, 2))


# speedup vs baseline: 1.2783x; 1.2783x over previous
import jax
import jax.numpy as jnp
from jax.experimental import pallas as pl
from jax.experimental.pallas import tpu as pltpu


def _pos_kernel(row_ref, col_ref, out_ref, scratch, sem):
    h = row_ref.shape[0]
    w = col_ref.shape[0]
    f_half = row_ref.shape[1]
    b = out_ref.shape[0]
    half = h // 2
    copies = []
    for lo in (0, half):
        scratch[lo:lo + half, :, 0:f_half] = jnp.broadcast_to(
            col_ref[...][None, :, :], (half, w, f_half)
        )
        scratch[lo:lo + half, :, f_half:2 * f_half] = jnp.broadcast_to(
            row_ref[lo:lo + half, :][:, None, :], (half, w, f_half)
        )
        for i in range(b):
            c = pltpu.make_async_copy(
                scratch.at[pl.ds(lo, half)],
                out_ref.at[i, pl.ds(lo, half)],
                sem,
            )
            c.start()
            copies.append(c)
    for c in copies:
        c.wait()


def kernel(mask, row_embed, col_embed):
    b, h, w = mask.shape
    f_half = row_embed.shape[1]
    f = 2 * f_half
    out = pl.pallas_call(
        _pos_kernel,
        out_specs=pl.BlockSpec(memory_space=pl.ANY),
        out_shape=jax.ShapeDtypeStruct((b, h, w, f), jnp.float32),
        scratch_shapes=[
            pltpu.VMEM((h, w, f), jnp.float32),
            pltpu.SemaphoreType.DMA,
        ],
    )(row_embed, col_embed)
    return jnp.transpose(out, (0, 3, 1, 2))


# speedup vs baseline: 1.2900x; 1.0092x over previous
import jax
import jax.numpy as jnp
from jax.experimental import pallas as pl
from jax.experimental.pallas import tpu as pltpu


def _pos_kernel(row_ref, col_ref, out_ref, scratch, sem):
    h = row_ref.shape[0]
    w = col_ref.shape[0]
    f_half = row_ref.shape[1]
    b = out_ref.shape[0]
    half = h // 4
    copies = []
    for lo in (0, half, 2 * half, 3 * half):
        scratch[lo:lo + half, :, 0:f_half] = jnp.broadcast_to(
            col_ref[...][None, :, :], (half, w, f_half)
        )
        scratch[lo:lo + half, :, f_half:2 * f_half] = jnp.broadcast_to(
            row_ref[lo:lo + half, :][:, None, :], (half, w, f_half)
        )
        for i in range(b):
            c = pltpu.make_async_copy(
                scratch.at[pl.ds(lo, half)],
                out_ref.at[i, pl.ds(lo, half)],
                sem,
            )
            c.start()
            copies.append(c)
    for c in copies:
        c.wait()


def kernel(mask, row_embed, col_embed):
    b, h, w = mask.shape
    f_half = row_embed.shape[1]
    f = 2 * f_half
    out = pl.pallas_call(
        _pos_kernel,
        out_specs=pl.BlockSpec(memory_space=pl.ANY),
        out_shape=jax.ShapeDtypeStruct((b, h, w, f), jnp.float32),
        scratch_shapes=[
            pltpu.VMEM((h, w, f), jnp.float32),
            pltpu.SemaphoreType.DMA,
        ],
    )(row_embed, col_embed)
    return jnp.transpose(out, (0, 3, 1, 2))
